# baseline (speedup 1.0000x reference)
import functools

import jax
import jax.numpy as jnp
from jax import lax
from jax.experimental import pallas as pl
from jax.experimental.pallas import tpu as pltpu
from jax.experimental.pallas import tpu_sc as plsc

D = 1024

NC, NS = 2, 16
NW = NC * NS

CHUNK = 64

TM = 1024
TN = 1024


def _embed_gather(idx_flat, tok_table):
    bt = idx_flat.shape[0]
    rows_per_w = bt // NW
    n_chunks = rows_per_w // CHUNK
    mesh = plsc.VectorSubcoreMesh(core_axis_name="c", subcore_axis_name="s")

    @functools.partial(
        pl.kernel,
        out_type=jax.ShapeDtypeStruct((bt, D), jnp.float32),
        mesh=mesh,
        scratch_types=[
            pltpu.VMEM((CHUNK,), jnp.int32),
            pltpu.VMEM((CHUNK, D), jnp.float32),
            pltpu.SemaphoreType.DMA,
        ],
    )
    def k(idx_hbm, tok_hbm, x_hbm, idx_v, rows_v, sem):
        wid = lax.axis_index("s") * NC + lax.axis_index("c")
        base = wid * rows_per_w
        for c in range(n_chunks):
            off = base + c * CHUNK
            pltpu.sync_copy(idx_hbm.at[pl.ds(off, CHUNK)], idx_v)
            pltpu.async_copy(tok_hbm.at[idx_v], rows_v, sem).wait()
            pltpu.sync_copy(rows_v, x_hbm.at[pl.ds(off, CHUNK)])

    return k(idx_flat, tok_table)


def _mm_body(x_ref, pos_ref, w_ref, b_ref, o_ref, xb_ref, xstage_ref, sem):
    n = pl.program_id(0)
    m = pl.program_id(1)
    t_tiles = pos_ref.shape[0] // TM

    @pl.when(n == 0)
    def _stage():
        pltpu.make_async_copy(
            x_ref.at[pl.ds(m * TM, TM), :], xstage_ref, sem).start()
        pltpu.make_async_copy(
            x_ref.at[pl.ds(m * TM, TM), :], xstage_ref, sem).wait()
        toff = (m % t_tiles) * TM
        xb_ref[pl.ds(m * TM, TM), :] = (
            xstage_ref[...] + pos_ref[pl.ds(toff, TM), :]
        ).astype(jnp.bfloat16)

    wb = w_ref[...].astype(jnp.bfloat16)
    acc = lax.dot_general(
        xb_ref[pl.ds(m * TM, TM), :], wb, (((1,), (1,)), ((), ())),
        preferred_element_type=jnp.float32,
    )
    o_ref[...] = acc + b_ref[...]


def _matmul(x, pos_table, W, b2):
    bt = x.shape[0]
    v = W.shape[0]
    t_len = pos_table.shape[0]
    return pl.pallas_call(
        _mm_body,
        grid=(v // TN, bt // TM),
        in_specs=[
            pl.BlockSpec(memory_space=pltpu.MemorySpace.HBM),
            pl.BlockSpec((t_len, D), lambda n, m: (0, 0)),
            pl.BlockSpec((TN, D), lambda n, m: (n, 0)),
            pl.BlockSpec((1, TN), lambda n, m: (0, n)),
        ],
        out_specs=pl.BlockSpec((TM, TN), lambda n, m: (m, n)),
        out_shape=jax.ShapeDtypeStruct((bt, v), jnp.float32),
        scratch_shapes=[
            pltpu.VMEM((bt, D), jnp.bfloat16),
            pltpu.VMEM((TM, D), jnp.float32),
            pltpu.SemaphoreType.DMA,
        ],
    )(x, pos_table, W, b2)


def kernel(idx, tok_table, pos_table, W, b):
    B, T = idx.shape
    v = W.shape[0]
    idx_flat = idx.reshape(-1).astype(jnp.int32)
    x = _embed_gather(idx_flat, tok_table)
    logits = _matmul(x, pos_table, W, b.reshape(1, -1))
    return logits.reshape(B, T, v)


# speedup vs baseline: 1.0636x; 1.0636x over previous
import functools

import jax
import jax.numpy as jnp
from jax import lax
from jax.experimental import pallas as pl
from jax.experimental.pallas import tpu as pltpu
from jax.experimental.pallas import tpu_sc as plsc

D = 1024

NC, NS = 2, 16
NW = NC * NS

CHUNK = 64

TM = 1024
TN = 1024


def _embed_gather(idx_flat, tok_table):
    bt = idx_flat.shape[0]
    rows_per_w = bt // NW
    n_chunks = rows_per_w // CHUNK
    mesh = plsc.VectorSubcoreMesh(core_axis_name="c", subcore_axis_name="s")

    @functools.partial(
        pl.kernel,
        out_type=jax.ShapeDtypeStruct((bt, D), jnp.float32),
        mesh=mesh,
        scratch_types=[
            pltpu.VMEM((CHUNK,), jnp.int32),
            pltpu.VMEM((CHUNK, D), jnp.float32),
            pltpu.SemaphoreType.DMA,
        ],
    )
    def k(idx_hbm, tok_hbm, x_hbm, idx_v, rows_v, sem):
        wid = lax.axis_index("s") * NC + lax.axis_index("c")
        base = wid * rows_per_w
        for c in range(n_chunks):
            off = base + c * CHUNK
            pltpu.sync_copy(idx_hbm.at[pl.ds(off, CHUNK)], idx_v)
            pltpu.async_copy(tok_hbm.at[idx_v], rows_v, sem).wait()
            pltpu.sync_copy(rows_v, x_hbm.at[pl.ds(off, CHUNK)])

    return k(idx_flat, tok_table)


def _mm_body(x_ref, pos_ref, w_ref, b_ref, o_ref, xb_ref):
    n = pl.program_id(0)
    m = pl.program_id(1)

    @pl.when(n == 0)
    def _stage():
        xb_ref[pl.ds(m * TM, TM), :] = (
            x_ref[...] + pos_ref[...]).astype(jnp.bfloat16)

    wb = w_ref[...].astype(jnp.bfloat16)
    acc = lax.dot_general(
        xb_ref[pl.ds(m * TM, TM), :], wb, (((1,), (1,)), ((), ())),
        preferred_element_type=jnp.float32,
    )
    o_ref[...] = acc + b_ref[...]


def _matmul(x, pos_table, W, b2):
    bt = x.shape[0]
    v = W.shape[0]
    t_len = pos_table.shape[0]
    t_tiles = t_len // TM
    return pl.pallas_call(
        _mm_body,
        grid=(v // TN, bt // TM),
        in_specs=[
            pl.BlockSpec((TM, D),
                         lambda n, m: (jnp.where(n == 0, m, 0), 0)),
            pl.BlockSpec((TM, D),
                         lambda n, m: (jnp.where(n == 0, m % t_tiles, 0), 0)),
            pl.BlockSpec((TN, D), lambda n, m: (n, 0)),
            pl.BlockSpec((1, TN), lambda n, m: (0, n)),
        ],
        out_specs=pl.BlockSpec((TM, TN), lambda n, m: (m, n)),
        out_shape=jax.ShapeDtypeStruct((bt, v), jnp.float32),
        scratch_shapes=[
            pltpu.VMEM((bt, D), jnp.bfloat16),
        ],
    )(x, pos_table, W, b2)


def kernel(idx, tok_table, pos_table, W, b):
    B, T = idx.shape
    v = W.shape[0]
    idx_flat = idx.reshape(-1).astype(jnp.int32)
    x = _embed_gather(idx_flat, tok_table)
    logits = _matmul(x, pos_table, W, b.reshape(1, -1))
    return logits.reshape(B, T, v)


# speedup vs baseline: 1.0669x; 1.0031x over previous
import functools

import jax
import jax.numpy as jnp
from jax import lax
from jax.experimental import pallas as pl
from jax.experimental.pallas import tpu as pltpu
from jax.experimental.pallas import tpu_sc as plsc

D = 1024

NC, NS = 2, 16
NW = NC * NS

CHUNK = 64

TM = 1024
TN = 2048


def _embed_gather(idx_flat, tok_table):
    bt = idx_flat.shape[0]
    rows_per_w = bt // NW
    n_chunks = rows_per_w // CHUNK
    mesh = plsc.VectorSubcoreMesh(core_axis_name="c", subcore_axis_name="s")

    @functools.partial(
        pl.kernel,
        out_type=jax.ShapeDtypeStruct((bt, D), jnp.float32),
        mesh=mesh,
        scratch_types=[
            pltpu.VMEM((CHUNK,), jnp.int32),
            pltpu.VMEM((CHUNK, D), jnp.float32),
            pltpu.SemaphoreType.DMA,
        ],
    )
    def k(idx_hbm, tok_hbm, x_hbm, idx_v, rows_v, sem):
        wid = lax.axis_index("s") * NC + lax.axis_index("c")
        base = wid * rows_per_w
        for c in range(n_chunks):
            off = base + c * CHUNK
            pltpu.sync_copy(idx_hbm.at[pl.ds(off, CHUNK)], idx_v)
            pltpu.async_copy(tok_hbm.at[idx_v], rows_v, sem).wait()
            pltpu.sync_copy(rows_v, x_hbm.at[pl.ds(off, CHUNK)])

    return k(idx_flat, tok_table)


def _mm_body(x_ref, pos_ref, w_ref, b_ref, o_ref, xsb_ref):
    n = pl.program_id(1)

    @pl.when(n == 0)
    def _stage():
        xsb_ref[...] = (x_ref[...] + pos_ref[...]).astype(jnp.bfloat16)

    acc = lax.dot_general(
        xsb_ref[...], w_ref[pl.ds(n * TN, TN), :], (((1,), (1,)), ((), ())),
        preferred_element_type=jnp.float32,
    )
    o_ref[...] = acc + b_ref[...]


def _matmul(x, pos_table, w_bf16, b2):
    bt = x.shape[0]
    v = w_bf16.shape[0]
    t_len = pos_table.shape[0]
    t_tiles = t_len // TM
    return pl.pallas_call(
        _mm_body,
        grid=(bt // TM, v // TN),
        in_specs=[
            pl.BlockSpec((TM, D), lambda m, n: (m, 0)),
            pl.BlockSpec((TM, D), lambda m, n: (m % t_tiles, 0)),
            pl.BlockSpec((v, D), lambda m, n: (0, 0)),
            pl.BlockSpec((1, TN), lambda m, n: (0, n)),
        ],
        out_specs=pl.BlockSpec((TM, TN), lambda m, n: (m, n)),
        out_shape=jax.ShapeDtypeStruct((bt, v), jnp.float32),
        scratch_shapes=[
            pltpu.VMEM((TM, D), jnp.bfloat16),
        ],
    )(x, pos_table, w_bf16, b2)


def kernel(idx, tok_table, pos_table, W, b):
    B, T = idx.shape
    v = W.shape[0]
    idx_flat = idx.reshape(-1).astype(jnp.int32)
    x = _embed_gather(idx_flat, tok_table)
    w_bf16 = W.astype(jnp.bfloat16)
    logits = _matmul(x, pos_table, w_bf16, b.reshape(1, -1))
    return logits.reshape(B, T, v)


# speedup vs baseline: 1.0799x; 1.0122x over previous
import functools

import jax
import jax.numpy as jnp
from jax import lax
from jax.experimental import pallas as pl
from jax.experimental.pallas import tpu as pltpu
from jax.experimental.pallas import tpu_sc as plsc

D = 1024

NC, NS = 2, 16
NW = NC * NS

CHUNK = 64

TM = 1024
TN = 1024


def _embed_gather(idx_flat, tok_table):
    bt = idx_flat.shape[0]
    rows_per_w = bt // NW
    n_chunks = rows_per_w // CHUNK
    mesh = plsc.VectorSubcoreMesh(core_axis_name="c", subcore_axis_name="s")

    @functools.partial(
        pl.kernel,
        out_type=jax.ShapeDtypeStruct((bt, D), jnp.float32),
        mesh=mesh,
        scratch_types=[
            pltpu.VMEM((CHUNK,), jnp.int32),
            pltpu.VMEM((CHUNK, D), jnp.float32),
            pltpu.SemaphoreType.DMA,
        ],
    )
    def k(idx_hbm, tok_hbm, x_hbm, idx_v, rows_v, sem):
        wid = lax.axis_index("s") * NC + lax.axis_index("c")
        base = wid * rows_per_w
        for c in range(n_chunks):
            off = base + c * CHUNK
            pltpu.sync_copy(idx_hbm.at[pl.ds(off, CHUNK)], idx_v)
            pltpu.async_copy(tok_hbm.at[idx_v], rows_v, sem).wait()
            pltpu.sync_copy(rows_v, x_hbm.at[pl.ds(off, CHUNK)])

    return k(idx_flat, tok_table)


def _mm_body(x_ref, pos_ref, w_ref, b_ref, o_ref):
    n = pl.program_id(0)
    m = pl.program_id(1)
    t_tiles = pos_ref.shape[0] // TM
    toff = (m % t_tiles) * TM
    xs = (x_ref[pl.ds(m * TM, TM), :]
          + pos_ref[pl.ds(toff, TM), :]).astype(jnp.bfloat16)
    acc = lax.dot_general(
        xs, w_ref[...], (((1,), (1,)), ((), ())),
        preferred_element_type=jnp.float32,
    )
    o_ref[...] = acc + b_ref[:, pl.ds(n * TN, TN)]


def _matmul(x, pos_table, w_bf16, b2):
    bt = x.shape[0]
    v = w_bf16.shape[0]
    t_len = pos_table.shape[0]
    return pl.pallas_call(
        _mm_body,
        grid=(v // TN, bt // TM),
        in_specs=[
            pl.BlockSpec((bt, D), lambda n, m: (0, 0)),
            pl.BlockSpec((t_len, D), lambda n, m: (0, 0)),
            pl.BlockSpec((TN, D), lambda n, m: (n, 0)),
            pl.BlockSpec((1, v), lambda n, m: (0, 0)),
        ],
        out_specs=pl.BlockSpec((TM, TN), lambda n, m: (m, n)),
        out_shape=jax.ShapeDtypeStruct((bt, v), jnp.float32),
    )(x, pos_table, w_bf16, b2)


def kernel(idx, tok_table, pos_table, W, b):
    B, T = idx.shape
    v = W.shape[0]
    idx_flat = idx.reshape(-1).astype(jnp.int32)
    x = _embed_gather(idx_flat, tok_table)
    w_bf16 = W.astype(jnp.bfloat16)
    logits = _matmul(x, pos_table, w_bf16, b.reshape(1, -1))
    return logits.reshape(B, T, v)


# speedup vs baseline: 1.1888x; 1.1009x over previous
import functools

import jax
import jax.numpy as jnp
from jax import lax
from jax.experimental import pallas as pl
from jax.experimental.pallas import tpu as pltpu
from jax.experimental.pallas import tpu_sc as plsc

D = 1024

NC, NS = 2, 16
NW = NC * NS

CHUNK = 64

TM = 512


def _embed_gather(idx_flat, tok_table):
    bt = idx_flat.shape[0]
    rows_per_w = bt // NW
    n_chunks = rows_per_w // CHUNK
    mesh = plsc.VectorSubcoreMesh(core_axis_name="c", subcore_axis_name="s")

    @functools.partial(
        pl.kernel,
        out_type=jax.ShapeDtypeStruct((bt, D), jnp.float32),
        mesh=mesh,
        scratch_types=[
            pltpu.VMEM((CHUNK,), jnp.int32),
            pltpu.VMEM((CHUNK, D), jnp.float32),
            pltpu.SemaphoreType.DMA,
        ],
    )
    def k(idx_hbm, tok_hbm, x_hbm, idx_v, rows_v, sem):
        wid = lax.axis_index("s") * NC + lax.axis_index("c")
        base = wid * rows_per_w
        for c in range(n_chunks):
            off = base + c * CHUNK
            pltpu.sync_copy(idx_hbm.at[pl.ds(off, CHUNK)], idx_v)
            pltpu.async_copy(tok_hbm.at[idx_v], rows_v, sem).wait()
            pltpu.sync_copy(rows_v, x_hbm.at[pl.ds(off, CHUNK)])

    return k(idx_flat, tok_table)


def _mm_body(x_ref, pos_ref, w_ref, b_ref, o_ref):
    xs = (x_ref[...] + pos_ref[...]).astype(jnp.bfloat16)
    acc = lax.dot_general(
        xs, w_ref[...], (((1,), (1,)), ((), ())),
        preferred_element_type=jnp.float32,
    )
    o_ref[...] = acc + b_ref[...]


def _matmul(x, pos_table, w_bf16, b2):
    bt = x.shape[0]
    v = w_bf16.shape[0]
    t_len = pos_table.shape[0]
    t_tiles = t_len // TM
    return pl.pallas_call(
        _mm_body,
        grid=(bt // TM,),
        in_specs=[
            pl.BlockSpec((TM, D), lambda m: (m, 0)),
            pl.BlockSpec((TM, D), lambda m: (m % t_tiles, 0)),
            pl.BlockSpec((v, D), lambda m: (0, 0)),
            pl.BlockSpec((1, v), lambda m: (0, 0)),
        ],
        out_specs=pl.BlockSpec((TM, v), lambda m: (m, 0)),
        out_shape=jax.ShapeDtypeStruct((bt, v), jnp.float32),
    )(x, pos_table, w_bf16, b2)


def kernel(idx, tok_table, pos_table, W, b):
    B, T = idx.shape
    v = W.shape[0]
    idx_flat = idx.reshape(-1).astype(jnp.int32)
    x = _embed_gather(idx_flat, tok_table)
    w_bf16 = W.astype(jnp.bfloat16)
    logits = _matmul(x, pos_table, w_bf16, b.reshape(1, -1))
    return logits.reshape(B, T, v)
